# baseline (speedup 1.0000x reference)
import functools

import jax
import jax.numpy as jnp
from jax import lax
from jax.experimental import pallas as pl
from jax.experimental.pallas import tpu as pltpu
from jax.experimental.pallas import tpu_sc as plsc

T = 2048
D = 1024
F = 2048
E = 16
S = 2 * T
BR = 128
NBLK = 48
RCAP = NBLK * BR
NW = 32
NEG = -1e30


def _gating_body(x_ref, gw_ref, gb_ref, idx_ref, sck_ref, pos_ref, meta_ref):
    x = x_ref[...]
    logits = jnp.dot(x, gw_ref[...], preferred_element_type=jnp.float32)
    logits = logits + gb_ref[...]
    iota = lax.broadcasted_iota(jnp.int32, (T, E), 1)
    m1 = jnp.max(logits, axis=1, keepdims=True)
    am1 = jnp.min(jnp.where(logits == m1, iota, E), axis=1, keepdims=True)
    oh1 = (iota == am1)
    l2 = jnp.where(oh1, NEG, logits)
    m2 = jnp.max(l2, axis=1, keepdims=True)
    am2 = jnp.min(jnp.where(l2 == m2, iota, E), axis=1, keepdims=True)
    oh2 = (iota == am2)
    t = jnp.exp(m2 - m1)
    denom = 1.0 + t
    idx_ref[...] = jnp.concatenate([am1, am2], axis=1)
    sck_ref[...] = jnp.concatenate([1.0 / denom, t / denom], axis=0)

    Z = jnp.concatenate([oh1.astype(jnp.float32), oh2.astype(jnp.float32)], axis=0)
    C = 512
    tri = (lax.broadcasted_iota(jnp.int32, (C, C), 0)
           > lax.broadcasted_iota(jnp.int32, (C, C), 1)).astype(jnp.float32)
    run = jnp.zeros((1, E), jnp.float32)
    ranks = []
    for c in range(S // C):
        Zc = Z[c * C:(c + 1) * C]
        excl = jnp.dot(tri, Zc, preferred_element_type=jnp.float32) + run
        ranks.append(jnp.sum(excl * Zc, axis=1, keepdims=True))
        run = run + jnp.sum(Zc, axis=0, keepdims=True)
    counts = run.astype(jnp.int32)
    aligned = ((counts + (BR - 1)) // BR) * BR
    U = (lax.broadcasted_iota(jnp.int32, (E, E), 0)
         < lax.broadcasted_iota(jnp.int32, (E, E), 1)).astype(jnp.float32)
    offs = jnp.dot(aligned.astype(jnp.float32), U,
                   preferred_element_type=jnp.float32)
    pos = []
    for c in range(S // C):
        Zc = Z[c * C:(c + 1) * C]
        off_sel = jnp.sum(Zc * offs, axis=1, keepdims=True)
        pos.append((ranks[c] + off_sel).astype(jnp.int32))
    pos_ref[...] = jnp.concatenate(pos, axis=0)

    total = jnp.sum(aligned)
    nblk = total // BR
    bstart = (lax.broadcasted_iota(jnp.int32, (NBLK, 1), 0) * BR).astype(jnp.float32)
    be = jnp.sum((offs <= bstart).astype(jnp.int32), axis=1, keepdims=True) - 1
    r = lax.broadcasted_iota(jnp.int32, (16, 1), 0)
    extra = jnp.where(r == 0, nblk, 0)
    meta_ref[...] = jnp.concatenate([be, extra], axis=0)


def _gating(x, gate_w, gate_b):
    return pl.pallas_call(
        _gating_body,
        out_shape=(
            jax.ShapeDtypeStruct((T, 2), jnp.int32),
            jax.ShapeDtypeStruct((S, 1), jnp.float32),
            jax.ShapeDtypeStruct((S, 1), jnp.int32),
            jax.ShapeDtypeStruct((NBLK + 16, 1), jnp.int32),
        ),
    )(x, gate_w, gate_b.reshape(1, E))


_APW = S // NW
_GCH = 64
_NCH = RCAP // (NW * _GCH)
_TPW = T // NW


@functools.cache
def _sc_kernels():
    mesh = plsc.VectorSubcoreMesh(core_axis_name="c", subcore_axis_name="s")

    @functools.partial(
        pl.kernel,
        out_type=(
            jax.ShapeDtypeStruct((RCAP,), jnp.int32),
            jax.ShapeDtypeStruct((RCAP,), jnp.float32),
        ),
        mesh=mesh,
        scratch_types=[
            pltpu.VMEM((_APW,), jnp.int32),
            pltpu.VMEM((_APW,), jnp.float32),
            pltpu.VMEM((_APW,), jnp.int32),
            pltpu.SemaphoreType.DMA,
        ],
    )
    def sc_scatter(pos_hbm, sc_hbm, tok_out, sco_out, posb, scob, tokb, sem):
        w = lax.axis_index("s") * 2 + lax.axis_index("c")
        base = w * _APW
        pltpu.sync_copy(pos_hbm.at[pl.ds(base, _APW)], posb)
        pltpu.sync_copy(sc_hbm.at[pl.ds(base, _APW)], scob)
        tbase = lax.rem(base, T)
        for i in range(_APW // 16):
            tokb[pl.ds(i * 16, 16)] = lax.iota(jnp.int32, 16) + (tbase + i * 16)
        pltpu.async_copy(tokb, tok_out.at[posb], sem).wait()
        pltpu.async_copy(scob, sco_out.at[posb], sem).wait()

    @functools.partial(
        pl.kernel,
        out_type=jax.ShapeDtypeStruct((RCAP, D), jnp.float32),
        mesh=mesh,
        scratch_types=[
            pltpu.VMEM((_GCH,), jnp.int32),
            pltpu.VMEM((_GCH, D), jnp.float32),
            pltpu.SemaphoreType.DMA,
        ],
    )
    def sc_gather_x(tok_hbm, x_hbm, xs_out, gidx, rows, sem):
        w = lax.axis_index("s") * 2 + lax.axis_index("c")
        for j in range(_NCH):
            r0 = (w * _NCH + j) * _GCH
            pltpu.sync_copy(tok_hbm.at[pl.ds(r0, _GCH)], gidx)
            for i in range(_GCH // 16):
                v = gidx[pl.ds(i * 16, 16)]
                gidx[pl.ds(i * 16, 16)] = jnp.minimum(jnp.maximum(v, 0), T - 1)
            pltpu.async_copy(x_hbm.at[gidx], rows, sem).wait()
            pltpu.sync_copy(rows, xs_out.at[pl.ds(r0, _GCH)])

    @functools.partial(
        pl.kernel,
        out_type=(
            jax.ShapeDtypeStruct((T, D), jnp.float32),
            jax.ShapeDtypeStruct((T, D), jnp.float32),
        ),
        mesh=mesh,
        scratch_types=[
            pltpu.VMEM((_TPW,), jnp.int32),
            pltpu.VMEM((_TPW, D), jnp.float32),
            pltpu.SemaphoreType.DMA,
        ],
    )
    def sc_combine(pos_hbm, ys_hbm, coa_out, cob_out, gidx, rows, sem):
        w = lax.axis_index("s") * 2 + lax.axis_index("c")
        t0 = w * _TPW
        pltpu.sync_copy(pos_hbm.at[pl.ds(t0, _TPW)], gidx)
        pltpu.async_copy(ys_hbm.at[gidx], rows, sem).wait()
        pltpu.sync_copy(rows, coa_out.at[pl.ds(t0, _TPW)])
        pltpu.sync_copy(pos_hbm.at[pl.ds(T + t0, _TPW)], gidx)
        pltpu.async_copy(ys_hbm.at[gidx], rows, sem).wait()
        pltpu.sync_copy(rows, cob_out.at[pl.ds(t0, _TPW)])

    return sc_scatter, sc_gather_x, sc_combine


def _ffn_body(meta_ref, xs_ref, w1_ref, b1_ref, w2_ref, b2_ref, sc_ref, out_ref):
    b = pl.program_id(0)

    @pl.when(b < meta_ref[NBLK])
    def _():
        h = jnp.dot(xs_ref[...], w1_ref[0], preferred_element_type=jnp.float32)
        h = jnp.maximum(h + b1_ref[0], 0.0)
        y = jnp.dot(h, w2_ref[0], preferred_element_type=jnp.float32)
        out_ref[...] = (y + b2_ref[0]) * sc_ref[...]


def _ffn(meta, xs, w1, b1, w2, b2, scs):
    grid_spec = pltpu.PrefetchScalarGridSpec(
        num_scalar_prefetch=1,
        grid=(NBLK,),
        in_specs=[
            pl.BlockSpec((BR, D), lambda b, m: (b, 0)),
            pl.BlockSpec((1, D, F), lambda b, m: (m[b], 0, 0)),
            pl.BlockSpec((1, 1, F), lambda b, m: (m[b], 0, 0)),
            pl.BlockSpec((1, F, D), lambda b, m: (m[b], 0, 0)),
            pl.BlockSpec((1, 1, D), lambda b, m: (m[b], 0, 0)),
            pl.BlockSpec((BR, 1), lambda b, m: (b, 0)),
        ],
        out_specs=pl.BlockSpec((BR, D), lambda b, m: (b, 0)),
    )
    return pl.pallas_call(
        _ffn_body,
        grid_spec=grid_spec,
        out_shape=jax.ShapeDtypeStruct((RCAP, D), jnp.float32),
    )(meta, xs, w1, b1.reshape(E, 1, F), w2, b2.reshape(E, 1, D), scs)


def _ln_body(x_ref, ca_ref, cb_ref, g_ref, b_ref, out_ref):
    v = x_ref[...] + ca_ref[...] + cb_ref[...]
    mu = jnp.mean(v, axis=1, keepdims=True)
    d = v - mu
    var = jnp.mean(d * d, axis=1, keepdims=True)
    out_ref[...] = d * lax.rsqrt(var + 1e-5) * g_ref[...] + b_ref[...]


def _ln(x, ca, cb, ln_g, ln_b):
    return pl.pallas_call(
        _ln_body,
        grid=(8,),
        in_specs=[
            pl.BlockSpec((T // 8, D), lambda i: (i, 0)),
            pl.BlockSpec((T // 8, D), lambda i: (i, 0)),
            pl.BlockSpec((T // 8, D), lambda i: (i, 0)),
            pl.BlockSpec((1, D), lambda i: (0, 0)),
            pl.BlockSpec((1, D), lambda i: (0, 0)),
        ],
        out_specs=pl.BlockSpec((T // 8, D), lambda i: (i, 0)),
        out_shape=jax.ShapeDtypeStruct((T, D), jnp.float32),
    )(x, ca, cb, ln_g.reshape(1, D), ln_b.reshape(1, D))


def kernel(inp, gate_w, gate_b, w1, b1, w2, b2, ln_g, ln_b):
    sc_scatter, sc_gather_x, sc_combine = _sc_kernels()
    topk_idx, sck, pos, meta = _gating(inp, gate_w, gate_b)
    pos1 = pos.reshape(S)
    tok_sorted, sc_sorted = sc_scatter(pos1, sck.reshape(S))
    xs = sc_gather_x(tok_sorted, inp)
    ys = _ffn(meta.reshape(NBLK + 16), xs, w1, b1, w2, b2,
              sc_sorted.reshape(RCAP, 1))
    coa, cob = sc_combine(pos1, ys)
    output = _ln(inp, coa, cob, ln_g, ln_b)
    return output, topk_idx


# speedup vs baseline: 1.7023x; 1.7023x over previous
import functools

import jax
import jax.numpy as jnp
from jax import lax
from jax.experimental import pallas as pl
from jax.experimental.pallas import tpu as pltpu
from jax.experimental.pallas import tpu_sc as plsc

T = 2048
D = 1024
F = 2048
E = 16
S = 2 * T
BR = 128
NBLK = 48
RCAP = NBLK * BR
NW = 32
NEG = -1e30


def _gating_body(x_ref, gw_ref, gb_ref, idx_ref, sa_ref, sb_ref, pos_ref, meta_ref):
    x = x_ref[...]
    logits = jnp.dot(x, gw_ref[...], preferred_element_type=jnp.float32)
    logits = logits + gb_ref[...]
    iota = lax.broadcasted_iota(jnp.int32, (T, E), 1)
    m1 = jnp.max(logits, axis=1, keepdims=True)
    am1 = jnp.min(jnp.where(logits == m1, iota, E), axis=1, keepdims=True)
    oh1 = (iota == am1)
    l2 = jnp.where(oh1, NEG, logits)
    m2 = jnp.max(l2, axis=1, keepdims=True)
    am2 = jnp.min(jnp.where(l2 == m2, iota, E), axis=1, keepdims=True)
    oh2 = (iota == am2)
    t = jnp.exp(m2 - m1)
    denom = 1.0 + t
    idx_ref[...] = jnp.concatenate([am1, am2], axis=1)
    sa_ref[...] = 1.0 / denom
    sb_ref[...] = t / denom

    Z = jnp.concatenate([oh1.astype(jnp.float32), oh2.astype(jnp.float32)], axis=0)
    C = 512
    tri = (lax.broadcasted_iota(jnp.int32, (C, C), 0)
           > lax.broadcasted_iota(jnp.int32, (C, C), 1)).astype(jnp.float32)
    run = jnp.zeros((1, E), jnp.float32)
    ranks = []
    for c in range(S // C):
        Zc = Z[c * C:(c + 1) * C]
        excl = jnp.dot(tri, Zc, preferred_element_type=jnp.float32) + run
        ranks.append(jnp.sum(excl * Zc, axis=1, keepdims=True))
        run = run + jnp.sum(Zc, axis=0, keepdims=True)
    counts = run.astype(jnp.int32)
    aligned = ((counts + (BR - 1)) // BR) * BR
    U = (lax.broadcasted_iota(jnp.int32, (E, E), 0)
         < lax.broadcasted_iota(jnp.int32, (E, E), 1)).astype(jnp.float32)
    offs = jnp.dot(aligned.astype(jnp.float32), U,
                   preferred_element_type=jnp.float32)
    pos = []
    for c in range(S // C):
        Zc = Z[c * C:(c + 1) * C]
        off_sel = jnp.sum(Zc * offs, axis=1, keepdims=True)
        pos.append((ranks[c] + off_sel).astype(jnp.int32))
    pos_ref[...] = jnp.concatenate(pos, axis=0)

    total = jnp.sum(aligned)
    nblk = total // BR
    bstart = (lax.broadcasted_iota(jnp.int32, (NBLK, 1), 0) * BR).astype(jnp.float32)
    be = jnp.sum((offs <= bstart).astype(jnp.int32), axis=1, keepdims=True) - 1
    r = lax.broadcasted_iota(jnp.int32, (16, 1), 0)
    extra = jnp.where(r == 0, nblk, 0)
    meta_ref[...] = jnp.concatenate([be, extra], axis=0)


def _gating(x, gate_w, gate_b):
    return pl.pallas_call(
        _gating_body,
        out_shape=(
            jax.ShapeDtypeStruct((T, 2), jnp.int32),
            jax.ShapeDtypeStruct((T, 1), jnp.float32),
            jax.ShapeDtypeStruct((T, 1), jnp.float32),
            jax.ShapeDtypeStruct((S, 1), jnp.int32),
            jax.ShapeDtypeStruct((NBLK + 16, 1), jnp.int32),
        ),
    )(x, gate_w, gate_b.reshape(1, E))


_APW = S // NW
_GCH = 64
_NCH = RCAP // (NW * _GCH)
_TPW = T // NW


@functools.cache
def _sc_kernels():
    mesh = plsc.VectorSubcoreMesh(core_axis_name="c", subcore_axis_name="s")

    @functools.partial(
        pl.kernel,
        out_type=jax.ShapeDtypeStruct((RCAP, D), jnp.float32),
        mesh=mesh,
        scratch_types=[
            pltpu.VMEM((_GCH,), jnp.int32),
            pltpu.VMEM((_GCH,), jnp.int32),
            pltpu.VMEM((_GCH, D), jnp.float32),
            pltpu.SemaphoreType.DMA,
        ],
    )
    def sc_disperse(pos_hbm, x_hbm, xs_out, pos0, pos1, rows, sem):
        w = lax.axis_index("s") * 2 + lax.axis_index("c")
        base = w * _APW
        tbase = lax.rem(base, T)
        pltpu.sync_copy(pos_hbm.at[pl.ds(base, _GCH)], pos0)
        pltpu.sync_copy(pos_hbm.at[pl.ds(base + _GCH, _GCH)], pos1)
        pltpu.sync_copy(x_hbm.at[pl.ds(tbase, _GCH)], rows)
        pltpu.async_copy(rows, xs_out.at[pos0], sem).wait()
        pltpu.sync_copy(x_hbm.at[pl.ds(tbase + _GCH, _GCH)], rows)
        pltpu.async_copy(rows, xs_out.at[pos1], sem).wait()

    @functools.partial(
        pl.kernel,
        out_type=(
            jax.ShapeDtypeStruct((T, D), jnp.float32),
            jax.ShapeDtypeStruct((T, D), jnp.float32),
        ),
        mesh=mesh,
        scratch_types=[
            pltpu.VMEM((_TPW,), jnp.int32),
            pltpu.VMEM((_TPW, D), jnp.float32),
            pltpu.SemaphoreType.DMA,
        ],
    )
    def sc_combine(pos_hbm, ys_hbm, coa_out, cob_out, gidx, rows, sem):
        w = lax.axis_index("s") * 2 + lax.axis_index("c")
        t0 = w * _TPW
        pltpu.sync_copy(pos_hbm.at[pl.ds(t0, _TPW)], gidx)
        pltpu.async_copy(ys_hbm.at[gidx], rows, sem).wait()
        pltpu.sync_copy(rows, coa_out.at[pl.ds(t0, _TPW)])
        pltpu.sync_copy(pos_hbm.at[pl.ds(T + t0, _TPW)], gidx)
        pltpu.async_copy(ys_hbm.at[gidx], rows, sem).wait()
        pltpu.sync_copy(rows, cob_out.at[pl.ds(t0, _TPW)])

    return sc_disperse, sc_combine


def _ffn_body(meta_ref, xs_ref, w1_ref, b1_ref, w2_ref, b2_ref, out_ref):
    b = pl.program_id(0)

    @pl.when(b < meta_ref[NBLK])
    def _():
        h = jnp.dot(xs_ref[...], w1_ref[0], preferred_element_type=jnp.float32)
        h = jnp.maximum(h + b1_ref[0], 0.0)
        y = jnp.dot(h, w2_ref[0], preferred_element_type=jnp.float32)
        out_ref[...] = y + b2_ref[0]


def _ffn(meta, xs, w1, b1, w2, b2):
    grid_spec = pltpu.PrefetchScalarGridSpec(
        num_scalar_prefetch=1,
        grid=(NBLK,),
        in_specs=[
            pl.BlockSpec((BR, D), lambda b, m: (b, 0)),
            pl.BlockSpec((1, D, F), lambda b, m: (m[b], 0, 0)),
            pl.BlockSpec((1, 1, F), lambda b, m: (m[b], 0, 0)),
            pl.BlockSpec((1, F, D), lambda b, m: (m[b], 0, 0)),
            pl.BlockSpec((1, 1, D), lambda b, m: (m[b], 0, 0)),
        ],
        out_specs=pl.BlockSpec((BR, D), lambda b, m: (b, 0)),
    )
    return pl.pallas_call(
        _ffn_body,
        grid_spec=grid_spec,
        out_shape=jax.ShapeDtypeStruct((RCAP, D), jnp.float32),
    )(meta, xs, w1, b1.reshape(E, 1, F), w2, b2.reshape(E, 1, D))


def _ln_body(x_ref, ca_ref, cb_ref, sa_ref, sb_ref, g_ref, b_ref, out_ref):
    v = x_ref[...] + sa_ref[...] * ca_ref[...] + sb_ref[...] * cb_ref[...]
    mu = jnp.mean(v, axis=1, keepdims=True)
    d = v - mu
    var = jnp.mean(d * d, axis=1, keepdims=True)
    out_ref[...] = d * lax.rsqrt(var + 1e-5) * g_ref[...] + b_ref[...]


def _ln(x, ca, cb, sa, sb, ln_g, ln_b):
    return pl.pallas_call(
        _ln_body,
        grid=(8,),
        in_specs=[
            pl.BlockSpec((T // 8, D), lambda i: (i, 0)),
            pl.BlockSpec((T // 8, D), lambda i: (i, 0)),
            pl.BlockSpec((T // 8, D), lambda i: (i, 0)),
            pl.BlockSpec((T // 8, 1), lambda i: (i, 0)),
            pl.BlockSpec((T // 8, 1), lambda i: (i, 0)),
            pl.BlockSpec((1, D), lambda i: (0, 0)),
            pl.BlockSpec((1, D), lambda i: (0, 0)),
        ],
        out_specs=pl.BlockSpec((T // 8, D), lambda i: (i, 0)),
        out_shape=jax.ShapeDtypeStruct((T, D), jnp.float32),
    )(x, ca, cb, sa, sb, ln_g.reshape(1, D), ln_b.reshape(1, D))


def kernel(inp, gate_w, gate_b, w1, b1, w2, b2, ln_g, ln_b):
    sc_disperse, sc_combine = _sc_kernels()
    topk_idx, sa, sb, pos, meta = _gating(inp, gate_w, gate_b)
    pos1 = pos.reshape(S)
    xs = sc_disperse(pos1, inp)
    ys = _ffn(meta.reshape(NBLK + 16), xs, w1, b1, w2, b2)
    coa, cob = sc_combine(pos1, ys)
    output = _ln(inp, coa, cob, sa, sb, ln_g, ln_b)
    return output, topk_idx
